# baseline (speedup 1.0000x reference)
import functools

import jax
import jax.numpy as jnp
from jax import lax
from jax.experimental import pallas as pl
from jax.experimental.pallas import tpu as pltpu
from jax.experimental.pallas import tpu_sc as plsc

N = 10000
NP = 10240
E = 160000
EP = 163840
D = 256
DH = 128
CHUNK = 64
NSUB = 16
ROWS_PER_SUB = NP // NSUB
CH_PER_SUB = EP // NSUB // CHUNK
NIDX = 40
NSLOT = 4
MBLK = 1024
GRID_M = NP // MBLK

def _dinv(deg):
    return jnp.where(deg > 0, lax.rsqrt(deg), 0.0)


@functools.lru_cache(maxsize=None)
def _sc_kernels():
    mesh = plsc.VectorSubcoreMesh(core_axis_name="c", subcore_axis_name="s")
    deg_kernel = _make_deg_kernel(mesh)
    scatter_kernel = _make_sc_scatter(mesh)
    return deg_kernel, scatter_kernel


def _make_deg_kernel(mesh):
  @functools.partial(
    pl.kernel,
    mesh=mesh,
    out_type=jax.ShapeDtypeStruct((NP,), jnp.float32),
    scratch_types=[
        pltpu.VMEM((CH_PER_SUB, CHUNK), jnp.int32),
        pltpu.VMEM((CHUNK,), jnp.float32),
        pltpu.VMEM((ROWS_PER_SUB,), jnp.float32),
        pltpu.VMEM_SHARED((NP,), jnp.float32),
    ],
)
  def _deg_kernel(row_hbm, deg_hbm, rowv, onesv, zerov, dacc):
    cid = lax.axis_index("c")
    sid = lax.axis_index("s")
    ones16 = jnp.ones((16,), jnp.float32)
    zero16 = jnp.zeros((16,), jnp.float32)
    for j in range(CHUNK // 16):
        onesv[pl.ds(j * 16, 16)] = ones16

    def zbody(i, _):
        zerov[pl.ds(i * 16, 16)] = zero16
        return 0

    lax.fori_loop(0, ROWS_PER_SUB // 16, zbody, 0)
    pltpu.sync_copy(zerov, dacc.at[pl.ds(sid * ROWS_PER_SUB, ROWS_PER_SUB)])
    plsc.subcore_barrier()

    pltpu.sync_copy(row_hbm.at[pl.ds(sid * CH_PER_SUB, CH_PER_SUB)], rowv)

    def body(j, _):
        pltpu.sync_copy(onesv, dacc.at[rowv.at[j]], add=True)
        return 0

    lax.fori_loop(0, CH_PER_SUB, body, 0)
    plsc.subcore_barrier()
    w = sid * 2 + cid
    npw = NP // (2 * NSUB)
    pltpu.sync_copy(dacc.at[pl.ds(w * npw, npw)], zerov.at[pl.ds(0, npw)])
    pltpu.sync_copy(zerov.at[pl.ds(0, npw)], deg_hbm.at[pl.ds(w * npw, npw)])

  return _deg_kernel


def _make_sc_scatter(mesh):
  @functools.partial(
    pl.kernel,
    mesh=mesh,
    out_type=[
        jax.ShapeDtypeStruct((NP, DH), jnp.float32),
        jax.ShapeDtypeStruct((NP, DH), jnp.float32),
    ],
    scratch_types=[
        pltpu.VMEM((NIDX, CHUNK), jnp.int32),
        pltpu.VMEM((NIDX, CHUNK), jnp.int32),
        pltpu.VMEM((NSLOT, CHUNK, DH), jnp.float32),
        pltpu.VMEM_SHARED((NP, DH), jnp.float32),
        pltpu.SemaphoreType.DMA,
        pltpu.SemaphoreType.DMA,
        pltpu.SemaphoreType.DMA,
        pltpu.SemaphoreType.DMA,
    ],
)
  def _sc_scatter(ga_hbm, gb_hbm, row_hbm, col_hbm, sa_hbm, sb_hbm,
                  rowv, colv, buf, acc, sem0, sem1, sem2, sem3):
    cid = lax.axis_index("c")
    sid = lax.axis_index("s")
    sems = [sem0, sem1, sem2, sem3]
    zero16 = jnp.zeros((16,), jnp.float32)

    def zrow(i, _):
        for j in range(DH // 16):
            buf[0, i, pl.ds(j * 16, 16)] = zero16
        return 0

    lax.fori_loop(0, CHUNK, zrow, 0)
    for t in range(ROWS_PER_SUB // CHUNK):
        pltpu.sync_copy(
            buf.at[0], acc.at[pl.ds(sid * ROWS_PER_SUB + t * CHUNK, CHUNK)]
        )
    plsc.subcore_barrier()

    def stage(gref, sg):
        cbase = sid * CH_PER_SUB + sg * NIDX
        pltpu.sync_copy(row_hbm.at[pl.ds(cbase, NIDX)], rowv)
        pltpu.sync_copy(col_hbm.at[pl.ds(cbase, NIDX)], colv)

        def body(t, _):
            for b in range(NSLOT):
                jj = NSLOT * t + b

                @pl.when(t > 0)
                def _(b=b, jj=jj):
                    pltpu.make_async_copy(
                        buf.at[b], acc.at[colv.at[jj - NSLOT]], sems[b]
                    ).wait()

                pltpu.async_copy(gref.at[rowv.at[jj]], buf.at[b], sems[b])
            for b in range(NSLOT):
                jj = NSLOT * t + b
                pltpu.make_async_copy(gref.at[rowv.at[jj]], buf.at[b], sems[b]).wait()
                pltpu.async_copy(buf.at[b], acc.at[colv.at[jj]], sems[b], add=True)
            return 0

        lax.fori_loop(0, NIDX // NSLOT, body, 0)
        for b in range(NSLOT):
            pltpu.make_async_copy(
                buf.at[b], acc.at[colv.at[NIDX - NSLOT + b]], sems[b]
            ).wait()

    @pl.when(cid == 0)
    def _():
        for sg in range(CH_PER_SUB // NIDX):
            stage(ga_hbm, sg)

    @pl.when(cid == 1)
    def _():
        for sg in range(CH_PER_SUB // NIDX):
            stage(gb_hbm, sg)

    plsc.subcore_barrier()
    base = sid * ROWS_PER_SUB

    def writeback(dst_hbm):
        for t in range(ROWS_PER_SUB // CHUNK):
            pltpu.sync_copy(acc.at[pl.ds(base + t * CHUNK, CHUNK)], buf.at[0])
            pltpu.sync_copy(buf.at[0], dst_hbm.at[pl.ds(base + t * CHUNK, CHUNK)])

    @pl.when(cid == 0)
    def _():
        writeback(sa_hbm)

    @pl.when(cid == 1)
    def _():
        writeback(sb_hbm)

  return _sc_scatter


def _tc1_body(x_ref, deg_ref, w_ref, b_ref, ga_ref, gb_ref):
    dinv = _dinv(deg_ref[...])
    h = jnp.dot(x_ref[...], w_ref[...], preferred_element_type=jnp.float32)
    g = dinv * (h + b_ref[...])
    ga_ref[...] = g[:, :DH]
    gb_ref[...] = g[:, DH:]


def _tc_mid_body(sa_ref, sb_ref, deg_ref, w_ref, b_ref, ga_ref, gb_ref):
    dinv = _dinv(deg_ref[...])
    u = dinv * jnp.concatenate([sa_ref[...], sb_ref[...]], axis=1)
    h = jnp.dot(u, w_ref[...], preferred_element_type=jnp.float32)
    g = dinv * (h + b_ref[...])
    ga_ref[...] = g[:, :DH]
    gb_ref[...] = g[:, DH:]


def _tc_final_body(sa_ref, sb_ref, deg_ref, out_ref):
    dinv = _dinv(deg_ref[...])
    out_ref[...] = dinv * jnp.concatenate(
        [sa_ref[...], sb_ref[...]], axis=1
    )


_half_spec = pl.BlockSpec((MBLK, DH), lambda j: (j, 0))
_deg_spec = pl.BlockSpec((MBLK, 1), lambda j: (j, 0))
_w_spec = pl.BlockSpec((D, D), lambda j: (0, 0))
_b_spec = pl.BlockSpec((1, D), lambda j: (0, 0))
_half_ty = jax.ShapeDtypeStruct((NP, DH), jnp.float32)

_tc1 = pl.pallas_call(
    _tc1_body,
    grid=(GRID_M,),
    in_specs=[pl.BlockSpec((MBLK, D), lambda j: (j, 0)), _deg_spec, _w_spec, _b_spec],
    out_specs=[_half_spec, _half_spec],
    out_shape=[_half_ty, _half_ty],
)

_tc_mid = pl.pallas_call(
    _tc_mid_body,
    grid=(GRID_M,),
    in_specs=[_half_spec, _half_spec, _deg_spec, _w_spec, _b_spec],
    out_specs=[_half_spec, _half_spec],
    out_shape=[_half_ty, _half_ty],
)

_tc_final = pl.pallas_call(
    _tc_final_body,
    grid=(GRID_M,),
    in_specs=[_half_spec, _half_spec, _deg_spec],
    out_specs=pl.BlockSpec((MBLK, D), lambda j: (j, 0)),
    out_shape=jax.ShapeDtypeStruct((NP, D), jnp.float32),
)


def kernel(x, edge_index, W1, b1, W2, b2, W3, b3):
    row = edge_index[0].astype(jnp.int32)
    col = edge_index[1].astype(jnp.int32)
    pad = N + (jnp.arange(EP - E, dtype=jnp.int32) % (NP - N))
    row2d = jnp.concatenate([row, pad]).reshape(EP // CHUNK, CHUNK)
    col2d = jnp.concatenate([col, pad]).reshape(EP // CHUNK, CHUNK)
    x_p = jnp.zeros((NP, D), jnp.float32).at[:N].set(x)

    deg_kernel, sc_scatter = _sc_kernels()
    deg = deg_kernel(row2d).reshape(NP, 1)
    b1r, b2r, b3r = b1.reshape(1, D), b2.reshape(1, D), b3.reshape(1, D)
    ga, gb = _tc1(x_p, deg, W1, b1r)
    sa, sb = sc_scatter(ga, gb, row2d, col2d)
    ga, gb = _tc_mid(sa, sb, deg, W2, b2r)
    sa, sb = sc_scatter(ga, gb, row2d, col2d)
    ga, gb = _tc_mid(sa, sb, deg, W3, b3r)
    sa, sb = sc_scatter(ga, gb, row2d, col2d)
    full = _tc_final(sa, sb, deg)
    return full[:N]


# speedup vs baseline: 1.1170x; 1.1170x over previous
import functools

import jax
import jax.numpy as jnp
from jax import lax
from jax.experimental import pallas as pl
from jax.experimental.pallas import tpu as pltpu
from jax.experimental.pallas import tpu_sc as plsc

N = 10000
NP = 10240
E = 160000
EP = 163840
D = 256
DH = 128
CHUNK = 128
NSUB = 16
ROWS_PER_SUB = NP // NSUB
CH_PER_SUB = EP // NSUB // CHUNK
NIDX = 40
MBLK = 1024
GRID_M = NP // MBLK

def _dinv(deg):
    return jnp.where(deg > 0, lax.rsqrt(deg), 0.0)


@functools.lru_cache(maxsize=None)
def _sc_kernels():
    mesh = plsc.VectorSubcoreMesh(core_axis_name="c", subcore_axis_name="s")
    deg_kernel = _make_deg_kernel(mesh)
    scatter_kernel = _make_sc_scatter(mesh)
    return deg_kernel, scatter_kernel


def _make_deg_kernel(mesh):
  @functools.partial(
    pl.kernel,
    mesh=mesh,
    out_type=jax.ShapeDtypeStruct((NP,), jnp.float32),
    scratch_types=[
        pltpu.VMEM((CH_PER_SUB, CHUNK), jnp.int32),
        pltpu.VMEM((CHUNK,), jnp.float32),
        pltpu.VMEM((ROWS_PER_SUB,), jnp.float32),
        pltpu.VMEM_SHARED((NP,), jnp.float32),
    ],
)
  def _deg_kernel(row_hbm, deg_hbm, rowv, onesv, zerov, dacc):
    cid = lax.axis_index("c")
    sid = lax.axis_index("s")
    ones16 = jnp.ones((16,), jnp.float32)
    zero16 = jnp.zeros((16,), jnp.float32)
    for j in range(CHUNK // 16):
        onesv[pl.ds(j * 16, 16)] = ones16

    def zbody(i, _):
        zerov[pl.ds(i * 16, 16)] = zero16
        return 0

    lax.fori_loop(0, ROWS_PER_SUB // 16, zbody, 0)
    pltpu.sync_copy(zerov, dacc.at[pl.ds(sid * ROWS_PER_SUB, ROWS_PER_SUB)])
    plsc.subcore_barrier()

    pltpu.sync_copy(row_hbm.at[pl.ds(sid * CH_PER_SUB, CH_PER_SUB)], rowv)

    def body(j, _):
        pltpu.sync_copy(onesv, dacc.at[rowv.at[j]], add=True)
        return 0

    lax.fori_loop(0, CH_PER_SUB, body, 0)
    plsc.subcore_barrier()
    w = sid * 2 + cid
    npw = NP // (2 * NSUB)
    pltpu.sync_copy(dacc.at[pl.ds(w * npw, npw)], zerov.at[pl.ds(0, npw)])
    pltpu.sync_copy(zerov.at[pl.ds(0, npw)], deg_hbm.at[pl.ds(w * npw, npw)])

  return _deg_kernel


def _make_sc_scatter(mesh):
  @functools.partial(
    pl.kernel,
    mesh=mesh,
    out_type=[
        jax.ShapeDtypeStruct((NP, DH), jnp.float32),
        jax.ShapeDtypeStruct((NP, DH), jnp.float32),
    ],
    scratch_types=[
        pltpu.VMEM((NIDX, CHUNK), jnp.int32),
        pltpu.VMEM((NIDX, CHUNK), jnp.int32),
        pltpu.VMEM((2, CHUNK, DH), jnp.float32),
        pltpu.VMEM_SHARED((NP, DH), jnp.float32),
        pltpu.SemaphoreType.DMA,
        pltpu.SemaphoreType.DMA,
    ],
)
  def _sc_scatter(ga_hbm, gb_hbm, row_hbm, col_hbm, sa_hbm, sb_hbm,
                  rowv, colv, buf, acc, sem0, sem1):
    cid = lax.axis_index("c")
    sid = lax.axis_index("s")
    zero16 = jnp.zeros((16,), jnp.float32)

    def zrow(i, _):
        for j in range(DH // 16):
            buf[0, i, pl.ds(j * 16, 16)] = zero16
        return 0

    lax.fori_loop(0, CHUNK, zrow, 0)
    for t in range(ROWS_PER_SUB // CHUNK):
        pltpu.sync_copy(
            buf.at[0], acc.at[pl.ds(sid * ROWS_PER_SUB + t * CHUNK, CHUNK)]
        )
    plsc.subcore_barrier()

    def stage(gref, s):
        cbase = sid * CH_PER_SUB + s * NIDX
        pltpu.sync_copy(row_hbm.at[pl.ds(cbase, NIDX)], rowv)
        pltpu.sync_copy(col_hbm.at[pl.ds(cbase, NIDX)], colv)
        pltpu.async_copy(gref.at[rowv.at[0]], buf.at[0], sem0)

        def body(t, _):
            j = 2 * t
            pltpu.async_copy(gref.at[rowv.at[j + 1]], buf.at[1], sem1)
            pltpu.make_async_copy(gref.at[rowv.at[j]], buf.at[0], sem0).wait()
            pltpu.sync_copy(buf.at[0], acc.at[colv.at[j]], add=True)

            @pl.when(t + 1 < NIDX // 2)
            def _():
                pltpu.async_copy(gref.at[rowv.at[j + 2]], buf.at[0], sem0)

            pltpu.make_async_copy(gref.at[rowv.at[j + 1]], buf.at[1], sem1).wait()
            pltpu.sync_copy(buf.at[1], acc.at[colv.at[j + 1]], add=True)
            return 0

        lax.fori_loop(0, NIDX // 2, body, 0)

    @pl.when(cid == 0)
    def _():
        for s in range(CH_PER_SUB // NIDX):
            stage(ga_hbm, s)

    @pl.when(cid == 1)
    def _():
        for s in range(CH_PER_SUB // NIDX):
            stage(gb_hbm, s)

    plsc.subcore_barrier()
    base = sid * ROWS_PER_SUB

    def writeback(dst_hbm):
        for t in range(ROWS_PER_SUB // CHUNK):
            pltpu.sync_copy(acc.at[pl.ds(base + t * CHUNK, CHUNK)], buf.at[0])
            pltpu.sync_copy(buf.at[0], dst_hbm.at[pl.ds(base + t * CHUNK, CHUNK)])

    @pl.when(cid == 0)
    def _():
        writeback(sa_hbm)

    @pl.when(cid == 1)
    def _():
        writeback(sb_hbm)

  return _sc_scatter


def _tc1_body(x_ref, deg_ref, w_ref, b_ref, ga_ref, gb_ref):
    dinv = _dinv(deg_ref[...])
    h = jnp.dot(x_ref[...], w_ref[...], preferred_element_type=jnp.float32)
    g = dinv * (h + b_ref[...])
    ga_ref[...] = g[:, :DH]
    gb_ref[...] = g[:, DH:]


def _tc_mid_body(sa_ref, sb_ref, deg_ref, w_ref, b_ref, ga_ref, gb_ref):
    dinv = _dinv(deg_ref[...])
    u = dinv * jnp.concatenate([sa_ref[...], sb_ref[...]], axis=1)
    h = jnp.dot(u, w_ref[...], preferred_element_type=jnp.float32)
    g = dinv * (h + b_ref[...])
    ga_ref[...] = g[:, :DH]
    gb_ref[...] = g[:, DH:]


def _tc_final_body(sa_ref, sb_ref, deg_ref, out_ref):
    dinv = _dinv(deg_ref[...])
    out_ref[...] = dinv * jnp.concatenate(
        [sa_ref[...], sb_ref[...]], axis=1
    )


_half_spec = pl.BlockSpec((MBLK, DH), lambda j: (j, 0))
_deg_spec = pl.BlockSpec((MBLK, 1), lambda j: (j, 0))
_w_spec = pl.BlockSpec((D, D), lambda j: (0, 0))
_b_spec = pl.BlockSpec((1, D), lambda j: (0, 0))
_half_ty = jax.ShapeDtypeStruct((NP, DH), jnp.float32)

_tc1 = pl.pallas_call(
    _tc1_body,
    grid=(GRID_M,),
    in_specs=[pl.BlockSpec((MBLK, D), lambda j: (j, 0)), _deg_spec, _w_spec, _b_spec],
    out_specs=[_half_spec, _half_spec],
    out_shape=[_half_ty, _half_ty],
)

_tc_mid = pl.pallas_call(
    _tc_mid_body,
    grid=(GRID_M,),
    in_specs=[_half_spec, _half_spec, _deg_spec, _w_spec, _b_spec],
    out_specs=[_half_spec, _half_spec],
    out_shape=[_half_ty, _half_ty],
)

_fin_half_spec = pl.BlockSpec((N // GRID_M, DH), lambda j: (j, 0))
_fin_deg_spec = pl.BlockSpec((N // GRID_M, 1), lambda j: (j, 0))
_tc_final = pl.pallas_call(
    _tc_final_body,
    grid=(GRID_M,),
    in_specs=[_fin_half_spec, _fin_half_spec, _fin_deg_spec],
    out_specs=pl.BlockSpec((N // GRID_M, D), lambda j: (j, 0)),
    out_shape=jax.ShapeDtypeStruct((N, D), jnp.float32),
)


def kernel(x, edge_index, W1, b1, W2, b2, W3, b3):
    row = edge_index[0].astype(jnp.int32)
    col = edge_index[1].astype(jnp.int32)
    pad = N + (jnp.arange(EP - E, dtype=jnp.int32) % (NP - N))
    row2d = jnp.concatenate([row, pad]).reshape(EP // CHUNK, CHUNK)
    col2d = jnp.concatenate([col, pad]).reshape(EP // CHUNK, CHUNK)
    x_p = jnp.zeros((NP, D), jnp.float32).at[:N].set(x)

    deg_kernel, sc_scatter = _sc_kernels()
    deg = deg_kernel(row2d).reshape(NP, 1)
    b1r, b2r, b3r = b1.reshape(1, D), b2.reshape(1, D), b3.reshape(1, D)
    ga, gb = _tc1(x_p, deg, W1, b1r)
    sa, sb = sc_scatter(ga, gb, row2d, col2d)
    ga, gb = _tc_mid(sa, sb, deg, W2, b2r)
    sa, sb = sc_scatter(ga, gb, row2d, col2d)
    ga, gb = _tc_mid(sa, sb, deg, W3, b3r)
    sa, sb = sc_scatter(ga, gb, row2d, col2d)
    return _tc_final(sa, sb, deg)
